# baseline (speedup 1.0000x reference)
import functools

import jax
import jax.numpy as jnp
from jax import lax
from jax.experimental import pallas as pl
from jax.experimental.pallas import tpu as pltpu
from jax.experimental.pallas import tpu_sc as plsc

B = 16384
D = 4096
NW = 32
BPW = B // NW
W = 8
NCHUNK = BPW // W

_mesh = plsc.VectorSubcoreMesh(core_axis_name="c", subcore_axis_name="s")


@functools.partial(
    pl.kernel,
    mesh=_mesh,
    out_type=jax.ShapeDtypeStruct((B, D), jnp.float32),
    scratch_types=[
        pltpu.VMEM((NCHUNK, W), jnp.int32),
        pltpu.VMEM((W, D), jnp.float32),
        pltpu.VMEM((W, D), jnp.float32),
        pltpu.SemaphoreType.DMA,
        pltpu.SemaphoreType.DMA,
        pltpu.SemaphoreType.DMA,
        pltpu.SemaphoreType.DMA,
    ],
)
def _sc_gather(idx_hbm, table_hbm, out_hbm, idx_v, row0, row1,
               semg0, semg1, semw0, semw1):
    wid = lax.axis_index("s") * 2 + lax.axis_index("c")
    base = wid * BPW
    pltpu.sync_copy(idx_hbm.at[wid], idx_v)

    pltpu.async_copy(table_hbm.at[idx_v.at[0]], row0, semg0)
    pltpu.async_copy(table_hbm.at[idx_v.at[1]], row1, semg1)

    def body(c2, carry):
        c = c2 * 2
        pltpu.make_async_copy(table_hbm.at[idx_v.at[c]], row0, semg0).wait()
        pltpu.async_copy(row0, out_hbm.at[pl.ds(base + c * W, W)], semw0)
        pltpu.make_async_copy(table_hbm.at[idx_v.at[c + 1]], row1, semg1).wait()
        pltpu.async_copy(row1, out_hbm.at[pl.ds(base + (c + 1) * W, W)], semw1)
        pltpu.make_async_copy(row0, out_hbm.at[pl.ds(base + c * W, W)], semw0).wait()
        pltpu.async_copy(table_hbm.at[idx_v.at[c + 2]], row0, semg0)
        pltpu.make_async_copy(row1, out_hbm.at[pl.ds(base + (c + 1) * W, W)], semw1).wait()
        pltpu.async_copy(table_hbm.at[idx_v.at[c + 3]], row1, semg1)
        return carry

    lax.fori_loop(0, NCHUNK // 2 - 1, body, 0)

    c = NCHUNK - 2
    pltpu.make_async_copy(table_hbm.at[idx_v.at[c]], row0, semg0).wait()
    pltpu.async_copy(row0, out_hbm.at[pl.ds(base + c * W, W)], semw0)
    pltpu.make_async_copy(table_hbm.at[idx_v.at[c + 1]], row1, semg1).wait()
    pltpu.async_copy(row1, out_hbm.at[pl.ds(base + (c + 1) * W, W)], semw1)
    pltpu.make_async_copy(row0, out_hbm.at[pl.ds(base + c * W, W)], semw0).wait()
    pltpu.make_async_copy(row1, out_hbm.at[pl.ds(base + (c + 1) * W, W)], semw1).wait()


def kernel(positions, weight):
    shape = positions.shape
    idx = positions.reshape(NW, NCHUNK, W).astype(jnp.int32)
    out = _sc_gather(idx, weight)
    return out.reshape(*shape, D)


# speedup vs baseline: 1.0375x; 1.0375x over previous
import functools

import jax
import jax.numpy as jnp
from jax import lax
from jax.experimental import pallas as pl
from jax.experimental.pallas import tpu as pltpu
from jax.experimental.pallas import tpu_sc as plsc

B = 16384
D = 4096
NW = 32
BPW = B // NW
W = 8
NCHUNK = BPW // W

_mesh = plsc.VectorSubcoreMesh(core_axis_name="c", subcore_axis_name="s")


@functools.partial(
    pl.kernel,
    mesh=_mesh,
    out_type=jax.ShapeDtypeStruct((B, D), jnp.float32),
    scratch_types=[
        pltpu.VMEM((NCHUNK, W), jnp.int32),
        pltpu.VMEM((W, D), jnp.float32),
        pltpu.VMEM((W, D), jnp.float32),
        pltpu.VMEM((W, D), jnp.float32),
        pltpu.SemaphoreType.DMA,
        pltpu.SemaphoreType.DMA,
        pltpu.SemaphoreType.DMA,
        pltpu.SemaphoreType.DMA,
        pltpu.SemaphoreType.DMA,
        pltpu.SemaphoreType.DMA,
    ],
)
def _sc_gather(idx_hbm, table_hbm, out_hbm, idx_v, row0, row1, row2,
               semg0, semg1, semg2, semw0, semw1, semw2):
    bufs = (row0, row1, row2)
    semg = (semg0, semg1, semg2)
    semw = (semw0, semw1, semw2)
    wid = lax.axis_index("s") * 2 + lax.axis_index("c")
    base = wid * BPW
    pltpu.sync_copy(idx_hbm.at[wid], idx_v)

    def gather(c, b):
        pltpu.async_copy(table_hbm.at[idx_v.at[c]], bufs[b], semg[b])

    def wait_gather(c, b):
        pltpu.make_async_copy(table_hbm.at[idx_v.at[c]], bufs[b], semg[b]).wait()

    def write_sync(c, b):
        pltpu.sync_copy(bufs[b], out_hbm.at[pl.ds(base + c * W, W)])

    gather(0, 0)
    gather(1, 1)
    gather(2, 2)

    def body(c3, carry):
        for b in range(3):
            cb = c3 * 3 + b
            wait_gather(cb, b)
            write_sync(cb, b)
            gather(cb + 3, b)
        return carry

    lax.fori_loop(0, NCHUNK // 3 - 1, body, 0)

    wait_gather(60, 0)
    write_sync(60, 0)
    gather(63, 0)
    wait_gather(61, 1)
    write_sync(61, 1)
    wait_gather(62, 2)
    write_sync(62, 2)
    wait_gather(63, 0)
    write_sync(63, 0)


def kernel(positions, weight):
    shape = positions.shape
    idx = positions.reshape(NW, NCHUNK, W).astype(jnp.int32)
    out = _sc_gather(idx, weight)
    return out.reshape(*shape, D)
